# baseline (speedup 1.0000x reference)
import functools

import jax
import jax.numpy as jnp
from jax import lax
from jax.experimental import pallas as pl
from jax.experimental.pallas import tpu as pltpu
from jax.experimental.pallas import tpu_sc as plsc

_NUM_CLASS = 1000
_B = 1024
_L = 50
_NC = 2
_NS = 16
_NW = _NC * _NS
_BPW = _B // _NW

_mesh = plsc.VectorSubcoreMesh(core_axis_name="c", subcore_axis_name="s")


@functools.partial(
    pl.kernel,
    mesh=_mesh,
    compiler_params=pltpu.CompilerParams(
        needs_layout_passes=False,
        skip_device_barrier=True,
        disable_bounds_checks=True,
        disable_semaphore_checks=True,
    ),
    out_type=jax.ShapeDtypeStruct((_B, _L, _NUM_CLASS), jnp.float32),
    scratch_types=[
        pltpu.VMEM((_BPW * _L,), jnp.int32),
        pltpu.VMEM((_L, _NUM_CLASS), jnp.float32),
        pltpu.VMEM((_L, _NUM_CLASS), jnp.float32),
        pltpu.SemaphoreType.DMA,
        pltpu.SemaphoreType.DMA,
        pltpu.SemaphoreType.DMA,
    ],
)
def _sc_onehot(x_hbm, out_hbm, xbuf, buf0, buf1, sem0, sem1, semx):
    wid = lax.axis_index("s") * _NC + lax.axis_index("c")
    base = wid * _BPW
    lane = lax.broadcasted_iota(jnp.int32, (16,), 0)
    zeros16 = jnp.zeros((16,), jnp.float32)

    pltpu.async_copy(x_hbm.at[pl.ds(base * _L, _BPW * _L)], xbuf, semx).wait()

    def _zero_row(r, carry):
        rvec = jnp.full((16,), r, jnp.int32)
        for buf in (buf0, buf1):
            for j in range(62):
                buf[r, pl.ds(16 * j, 16)] = zeros16
            plsc.store_scatter(buf, [rvec, 992 + lane], zeros16,
                               mask=lane < 8)
        return carry

    lax.fori_loop(0, _L, _zero_row, 0)

    def _plant(b_local, buf, val):
        vals = jnp.full((16,), val, jnp.float32)
        for j in range(4):
            l_base = 16 * j if j < 3 else 34
            lidx = l_base + lane
            cidx = plsc.load_gather(xbuf, [b_local * _L + lidx])
            if j < 3:
                plsc.store_scatter(buf, [lidx, cidx], vals)
            else:
                plsc.store_scatter(buf, [lidx, cidx], vals, mask=lane >= 14)

    bufs = (buf0, buf1)
    sems = (sem0, sem1)

    def _body(t, carry):
        for k in range(2):
            i = 2 * t + k
            buf, sem = bufs[k], sems[k]

            @pl.when(t >= 1)
            def _():
                pltpu.make_async_copy(buf, out_hbm.at[base + i - 2], sem).wait()
                _plant(i - 2, buf, 0.0)

            _plant(i, buf, 1.0)
            pltpu.async_copy(buf, out_hbm.at[base + i], sem)
        return carry

    lax.fori_loop(0, _BPW // 2, _body, 0)

    for k in range(2):
        pltpu.make_async_copy(bufs[k], out_hbm.at[base + _BPW - 2 + k],
                              sems[k]).wait()


def kernel(x):
    xf = x.astype(jnp.int32).reshape(-1)
    return _sc_onehot(xf)


# speedup vs baseline: 4.9555x; 4.9555x over previous
import jax
import jax.numpy as jnp
from jax.experimental import pallas as pl

_NUM_CLASS = 1000
_CB = 40


def _body(xt_ref, o_ref):
    i = pl.program_id(0)
    xt = xt_ref[...]
    shape = (xt.shape[0], _CB, xt.shape[1])
    c = jax.lax.broadcasted_iota(jnp.int32, shape, 1) + i * _CB
    o_ref[...] = (c == xt[:, None, :]).astype(jnp.float32)


def kernel(x):
    B, L = x.shape
    xt = x.astype(jnp.int32).T
    out_t = pl.pallas_call(
        _body,
        grid=(_NUM_CLASS // _CB,),
        in_specs=[pl.BlockSpec((L, B), lambda i: (0, 0))],
        out_specs=pl.BlockSpec((L, _CB, B), lambda i: (0, i, 0)),
        out_shape=jax.ShapeDtypeStruct((L, _NUM_CLASS, B), jnp.float32),
    )(xt)
    return jnp.transpose(out_t, (2, 0, 1))
